# baseline (speedup 1.0000x reference)
import jax
import jax.numpy as jnp
from jax import lax
from jax.experimental import pallas as pl
from jax.experimental.pallas import tpu as pltpu
from jax.experimental.pallas import tpu_sc as plsc

_B, _T, _V, _L = 4, 256, 16, 8
_N = 40
_NC, _NS = 2, 16
_NW = _NC * _NS
_PAIRS_PER_W = (_B * _V) // _NW


_DNUMS = lax.GatherDimensionNumbers(
    offset_dims=(), collapsed_slice_dims=(0,), start_index_map=(0,))


def _take(vec, idx):
    return lax.gather(vec, idx[:, None], _DNUMS, (1,),
                      mode=lax.GatherScatterMode.PROMISE_IN_BOUNDS)


def _sc_body(srl_hbm, vn_hbm, vlab_hbm, sll_hbm, sem_hbm, vl_hbm, ol_hbm,
             out_hbm, vlab_v, sll_v, sem_v, vl_v, ol_v, rows_v, res_v, sem, sem2):
    wid = lax.axis_index("s") * _NC + lax.axis_index("c")
    iota = lax.iota(jnp.int32, 16)
    pair0 = wid * _PAIRS_PER_W
    b = pair0 // _V
    v0 = pair0 - b * _V

    stage_a = [
        pltpu.async_copy(vlab_hbm.at[b], vlab_v, sem2),
        pltpu.async_copy(sem_hbm.at[b], sem_v, sem2),
    ]
    stage_b = [
        pltpu.async_copy(sll_hbm.at[b], sll_v, sem),
        pltpu.async_copy(vl_hbm, vl_v.at[pl.ds(0, _B)], sem),
        pltpu.async_copy(ol_hbm, ol_v.at[pl.ds(0, _B)], sem),
    ]
    for c in stage_a:
        c.wait()

    j_u = iota // 8
    l_u = iota - j_u * 8
    v0vec = jnp.full((16,), v0, jnp.int32)
    sr0 = plsc.load_gather(sem_v, [j_u, l_u, v0vec])
    sr1 = plsc.load_gather(sem_v, [j_u, l_u, v0vec + 1])
    vlab0 = plsc.load_gather(vlab_v, [v0vec])[0]
    vlab1 = plsc.load_gather(vlab_v, [v0vec + 1])[0]

    rvec = jnp.where(iota < 8, _take(sr0, l_u), _take(sr1, l_u))
    avec = jnp.where(iota < 8, _take(sr0, l_u + 8), _take(sr1, l_u + 8))
    rm_u = (rvec != 0).astype(jnp.float32)
    am_u = (avec != 0).astype(jnp.float32)
    xoff_u = (iota + j_u * 8) * _T

    def fire_srl(u, _):
        uvec = jnp.full((16,), u, jnp.int32)
        r = _take(rvec, uvec)[0]
        off = pl.multiple_of(_take(xoff_u, uvec)[0], _T)
        vlab = jnp.where(u < 8, vlab0, vlab1)
        pltpu.async_copy(srl_hbm.at[b, vlab, r],
                         rows_v.at[pl.ds(off, _T)], sem)
        return 0

    def fire_vn(u, _):
        uvec = jnp.full((16,), u, jnp.int32)
        a = _take(avec, uvec)[0]
        off = pl.multiple_of(_take(xoff_u, uvec)[0], _T) + _L * _T
        vlab = jnp.where(u < 8, vlab0, vlab1)
        pltpu.async_copy(vn_hbm.at[b, vlab, a],
                         rows_v.at[pl.ds(off, _T)], sem)
        return 0

    lax.fori_loop(0, 16, fire_srl, 0)
    lax.fori_loop(0, 16, fire_vn, 0)

    for c in stage_b:
        c.wait()
    bvec = jnp.full((16,), b, jnp.int32)
    oln = plsc.load_gather(ol_v, [bvec])[0]
    vl_b = plsc.load_gather(vl_v, [bvec])[0]
    olv = ol_v[...]
    nrm = olv[0] + olv[1] + olv[2] + olv[3]
    inv_vec = 1.0 / jnp.full((16,), nrm.astype(jnp.float32))
    sll0 = plsc.load_gather(sll_v, [v0vec])[0]
    sll1 = plsc.load_gather(sll_v, [v0vec + 1])[0]
    sll_u = jnp.where(iota < 8, jnp.full((16,), sll0, jnp.int32),
                      jnp.full((16,), sll1, jnp.int32))
    use_u = (l_u < sll_u) & (v0 + j_u < vl_b) & ((rvec != 0) | (avec != 0))
    nfull = oln // 16
    cp = nfull
    has_p = (oln - nfull * 16) != 0
    trip_u = jnp.where(use_u, jnp.full((16,), nfull, jnp.int32), 0)
    pf_u = jnp.where(use_u & has_p, jnp.full((16,), 1.0, jnp.float32), 0.0)
    tmask_p = (cp * 16 + iota < oln).astype(jnp.float32)
    po = pl.multiple_of(cp * 16, 16)

    def drain(i, _):
        pltpu.make_async_copy(srl_hbm.at[0, 0, 0],
                              rows_v.at[pl.ds(0, _T)], sem).wait()
        return 0

    lax.fori_loop(0, 32, drain, 0)

    def u_body(u, tot):
        uvec = jnp.full((16,), u, jnp.int32)
        trip = _take(trip_u, uvec)[0]
        rm = _take(rm_u, uvec)[0]
        am = _take(am_u, uvec)[0]
        pf = _take(pf_u, uvec)[0]
        xo = pl.multiple_of(_take(xoff_u, uvec)[0], _T)
        yo = xo + _L * _T

        def chunk(c, acc):
            x = rows_v[pl.ds(xo + c * 16, 16)] * rm
            y = rows_v[pl.ds(yo + c * 16, 16)] * am
            return acc + jnp.abs(x - y)

        tot = lax.fori_loop(0, trip, chunk, tot)
        xp = rows_v[pl.ds(xo + po, 16)] * rm
        yp = rows_v[pl.ds(yo + po, 16)] * am
        return tot + jnp.abs(xp - yp) * (tmask_p * pf)

    total = lax.fori_loop(0, 16, u_body, jnp.zeros((16,), jnp.float32))

    res_v[...] = total * inv_vec
    pltpu.sync_copy(res_v, out_hbm.at[wid])


def kernel(log_srl, log_vn, v_label, v_l, orig_l, semlink, semlink_l):
    srl_t = jnp.transpose(log_srl, (0, 1, 3, 2))
    vn_t = jnp.transpose(log_vn, (0, 1, 3, 2))
    sem_t = jnp.transpose(semlink.astype(jnp.int32), (0, 2, 3, 1))

    sc_call = pl.kernel(
        _sc_body,
        out_type=jax.ShapeDtypeStruct((_NW, 16), jnp.float32),
        mesh=plsc.VectorSubcoreMesh(core_axis_name="c", subcore_axis_name="s"),
        scratch_types=[
            pltpu.VMEM((_V,), jnp.int32),
            pltpu.VMEM((_V,), jnp.int32),
            pltpu.VMEM((2, _L, _V), jnp.int32),
            pltpu.VMEM((16,), jnp.int32),
            pltpu.VMEM((16,), jnp.int32),
            pltpu.VMEM((2 * 2 * _L * _T + 16,), jnp.float32),
            pltpu.VMEM((16,), jnp.float32),
            pltpu.SemaphoreType.DMA,
            pltpu.SemaphoreType.DMA,
        ],
        compiler_params=pltpu.CompilerParams(
            needs_layout_passes=False,
            use_tc_tiling_on_sc=True,
        ),
    )
    partials = sc_call(srl_t, vn_t,
                       v_label.astype(jnp.int32), semlink_l.astype(jnp.int32),
                       sem_t, v_l.astype(jnp.int32), orig_l.astype(jnp.int32))
    return jnp.sum(partials)


# speedup vs baseline: 1.0151x; 1.0151x over previous
import jax
import jax.numpy as jnp
from jax import lax
from jax.experimental import pallas as pl
from jax.experimental.pallas import tpu as pltpu
from jax.experimental.pallas import tpu_sc as plsc

_B, _T, _V, _L = 4, 256, 16, 8
_N = 40
_NC, _NS = 2, 16
_NW = _NC * _NS
_PAIRS_PER_W = (_B * _V) // _NW


_DNUMS = lax.GatherDimensionNumbers(
    offset_dims=(), collapsed_slice_dims=(0,), start_index_map=(0,))


def _take(vec, idx):
    return lax.gather(vec, idx[:, None], _DNUMS, (1,),
                      mode=lax.GatherScatterMode.PROMISE_IN_BOUNDS)


def _sc_body(srl_hbm, vn_hbm, vlab_hbm, sll_hbm, sem_hbm, vl_hbm, ol_hbm,
             out_hbm, vlab_v, sll_v, sem_v, vl_v, ol_v, rows_v, res_v, sem, sem2):
    wid = lax.axis_index("s") * _NC + lax.axis_index("c")
    iota = lax.iota(jnp.int32, 16)
    pair0 = wid * _PAIRS_PER_W
    b = pair0 // _V
    v0 = pair0 - b * _V

    stage_a = [
        pltpu.async_copy(vlab_hbm.at[b], vlab_v, sem2),
        pltpu.async_copy(sem_hbm.at[b], sem_v, sem2),
    ]
    stage_b = [
        pltpu.async_copy(sll_hbm.at[b], sll_v, sem),
        pltpu.async_copy(vl_hbm, vl_v.at[pl.ds(0, _B)], sem),
        pltpu.async_copy(ol_hbm, ol_v.at[pl.ds(0, _B)], sem),
    ]
    for c in stage_a:
        c.wait()

    j_u = iota // 8
    l_u = iota - j_u * 8
    v0vec = jnp.full((16,), v0, jnp.int32)
    sr0 = plsc.load_gather(sem_v, [j_u, l_u, v0vec])
    sr1 = plsc.load_gather(sem_v, [j_u, l_u, v0vec + 1])
    vlab0 = plsc.load_gather(vlab_v, [v0vec])[0]
    vlab1 = plsc.load_gather(vlab_v, [v0vec + 1])[0]

    rvec = jnp.where(iota < 8, _take(sr0, l_u), _take(sr1, l_u))
    avec = jnp.where(iota < 8, _take(sr0, l_u + 8), _take(sr1, l_u + 8))
    rm_u = (rvec != 0).astype(jnp.float32)
    am_u = (avec != 0).astype(jnp.float32)
    xoff_u = (iota + j_u * 8) * _T

    def fire(u, _):
        uvec = jnp.full((16,), u, jnp.int32)
        r = _take(rvec, uvec)[0]
        a = _take(avec, uvec)[0]
        off = pl.multiple_of(_take(xoff_u, uvec)[0], _T)
        vlab = jnp.where(u < 8, vlab0, vlab1)
        pltpu.async_copy(srl_hbm.at[b, vlab, r],
                         rows_v.at[pl.ds(off, _T)], sem)
        pltpu.async_copy(vn_hbm.at[b, vlab, a],
                         rows_v.at[pl.ds(off + _L * _T, _T)], sem)
        return 0

    lax.fori_loop(0, 16, fire, 0)

    for c in stage_b:
        c.wait()
    bvec = jnp.full((16,), b, jnp.int32)
    oln = plsc.load_gather(ol_v, [bvec])[0]
    vl_b = plsc.load_gather(vl_v, [bvec])[0]
    olv = ol_v[...]
    nrm = olv[0] + olv[1] + olv[2] + olv[3]
    inv_vec = 1.0 / jnp.full((16,), nrm.astype(jnp.float32))
    sll0 = plsc.load_gather(sll_v, [v0vec])[0]
    sll1 = plsc.load_gather(sll_v, [v0vec + 1])[0]
    sll_u = jnp.where(iota < 8, jnp.full((16,), sll0, jnp.int32),
                      jnp.full((16,), sll1, jnp.int32))
    use_u = (l_u < sll_u) & (v0 + j_u < vl_b) & ((rvec != 0) | (avec != 0))
    nfull = oln // 16
    cp = nfull
    has_p = (oln - nfull * 16) != 0
    trip_u = jnp.where(use_u, jnp.full((16,), nfull, jnp.int32), 0)
    pf_u = jnp.where(use_u & has_p, jnp.full((16,), 1.0, jnp.float32), 0.0)
    tmask_p = (cp * 16 + iota < oln).astype(jnp.float32)
    po = pl.multiple_of(cp * 16, 16)

    def drain(i, _):
        pltpu.make_async_copy(srl_hbm.at[0, 0, 0],
                              rows_v.at[pl.ds(0, _T)], sem).wait()
        return 0

    lax.fori_loop(0, 32, drain, 0)

    def u_body(u, tot):
        uvec = jnp.full((16,), u, jnp.int32)
        trip = _take(trip_u, uvec)[0]
        rm = _take(rm_u, uvec)[0]
        am = _take(am_u, uvec)[0]
        pf = _take(pf_u, uvec)[0]
        xo = pl.multiple_of(_take(xoff_u, uvec)[0], _T)
        yo = xo + _L * _T

        def chunk2(c, acc):
            x0 = rows_v[pl.ds(xo + c * 32, 16)] * rm
            y0 = rows_v[pl.ds(yo + c * 32, 16)] * am
            x1 = rows_v[pl.ds(xo + c * 32 + 16, 16)] * rm
            y1 = rows_v[pl.ds(yo + c * 32 + 16, 16)] * am
            return acc + jnp.abs(x0 - y0) + jnp.abs(x1 - y1)

        tot = lax.fori_loop(0, trip // 2, chunk2, tot)
        lo = pl.multiple_of(jnp.maximum(trip - 1, 0) * 16, 16)
        lf = (trip - (trip // 2) * 2).astype(jnp.float32)
        xl = rows_v[pl.ds(xo + lo, 16)] * rm
        yl = rows_v[pl.ds(yo + lo, 16)] * am
        tot = tot + jnp.abs(xl - yl) * lf
        xp = rows_v[pl.ds(xo + po, 16)] * rm
        yp = rows_v[pl.ds(yo + po, 16)] * am
        return tot + jnp.abs(xp - yp) * (tmask_p * pf)

    total = lax.fori_loop(0, 16, u_body, jnp.zeros((16,), jnp.float32))

    res_v[...] = total * inv_vec
    pltpu.sync_copy(res_v, out_hbm.at[wid])


def kernel(log_srl, log_vn, v_label, v_l, orig_l, semlink, semlink_l):
    srl_t = jnp.transpose(log_srl, (0, 1, 3, 2))
    vn_t = jnp.transpose(log_vn, (0, 1, 3, 2))
    sem_t = jnp.transpose(semlink.astype(jnp.int32), (0, 2, 3, 1))

    sc_call = pl.kernel(
        _sc_body,
        out_type=jax.ShapeDtypeStruct((_NW, 16), jnp.float32),
        mesh=plsc.VectorSubcoreMesh(core_axis_name="c", subcore_axis_name="s"),
        scratch_types=[
            pltpu.VMEM((_V,), jnp.int32),
            pltpu.VMEM((_V,), jnp.int32),
            pltpu.VMEM((2, _L, _V), jnp.int32),
            pltpu.VMEM((16,), jnp.int32),
            pltpu.VMEM((16,), jnp.int32),
            pltpu.VMEM((2 * 2 * _L * _T + 16,), jnp.float32),
            pltpu.VMEM((16,), jnp.float32),
            pltpu.SemaphoreType.DMA,
            pltpu.SemaphoreType.DMA,
        ],
        compiler_params=pltpu.CompilerParams(
            needs_layout_passes=False,
            use_tc_tiling_on_sc=True,
        ),
    )
    partials = sc_call(srl_t, vn_t,
                       v_label.astype(jnp.int32), semlink_l.astype(jnp.int32),
                       sem_t, v_l.astype(jnp.int32), orig_l.astype(jnp.int32))
    return jnp.sum(partials)


# speedup vs baseline: 1.0263x; 1.0110x over previous
import jax
import jax.numpy as jnp
from jax import lax
from jax.experimental import pallas as pl
from jax.experimental.pallas import tpu as pltpu
from jax.experimental.pallas import tpu_sc as plsc

_B, _T, _V, _L = 4, 256, 16, 8
_N = 40
_NC, _NS = 2, 16
_NW = _NC * _NS
_PAIRS_PER_W = (_B * _V) // _NW


_DNUMS = lax.GatherDimensionNumbers(
    offset_dims=(), collapsed_slice_dims=(0,), start_index_map=(0,))


def _take(vec, idx):
    return lax.gather(vec, idx[:, None], _DNUMS, (1,),
                      mode=lax.GatherScatterMode.PROMISE_IN_BOUNDS)


def _sc_body(srl_hbm, vn_hbm, vlab_hbm, sll_hbm, sem_hbm, vl_hbm, ol_hbm,
             out_hbm, vlab_v, sll_v, sem_v, vl_v, ol_v, ids_v, rows_v, res_v, sem, sem2):
    wid = lax.axis_index("s") * _NC + lax.axis_index("c")
    iota = lax.iota(jnp.int32, 16)
    pair0 = wid * _PAIRS_PER_W
    b = pair0 // _V
    v0 = pair0 - b * _V

    stage_a = [
        pltpu.async_copy(vlab_hbm.at[b], vlab_v, sem2),
        pltpu.async_copy(sem_hbm.at[b], sem_v, sem2),
    ]
    stage_b = [
        pltpu.async_copy(sll_hbm.at[b], sll_v, sem),
        pltpu.async_copy(vl_hbm, vl_v.at[pl.ds(0, _B)], sem),
        pltpu.async_copy(ol_hbm, ol_v.at[pl.ds(0, _B)], sem),
    ]
    for c in stage_a:
        c.wait()

    j_u = iota // 8
    l_u = iota - j_u * 8
    v0vec = jnp.full((16,), v0, jnp.int32)
    sr0 = plsc.load_gather(sem_v, [j_u, l_u, v0vec])
    sr1 = plsc.load_gather(sem_v, [j_u, l_u, v0vec + 1])
    vlab0 = plsc.load_gather(vlab_v, [v0vec])[0]
    vlab1 = plsc.load_gather(vlab_v, [v0vec + 1])[0]

    rvec = jnp.where(iota < 8, _take(sr0, l_u), _take(sr1, l_u))
    avec = jnp.where(iota < 8, _take(sr0, l_u + 8), _take(sr1, l_u + 8))
    rm_u = (rvec != 0).astype(jnp.float32)
    am_u = (avec != 0).astype(jnp.float32)
    xoff_u = (iota + j_u * 8) * _T

    for c in stage_b:
        c.wait()
    bvec = jnp.full((16,), b, jnp.int32)
    oln = plsc.load_gather(ol_v, [bvec])[0]
    vl_b = plsc.load_gather(vl_v, [bvec])[0]
    olv = ol_v[...]
    nrm = olv[0] + olv[1] + olv[2] + olv[3]
    inv_vec = 1.0 / jnp.full((16,), nrm.astype(jnp.float32))
    sll0 = plsc.load_gather(sll_v, [v0vec])[0]
    sll1 = plsc.load_gather(sll_v, [v0vec + 1])[0]
    sll_u = jnp.where(iota < 8, jnp.full((16,), sll0, jnp.int32),
                      jnp.full((16,), sll1, jnp.int32))
    use_u = (l_u < sll_u) & (v0 + j_u < vl_b) & ((rvec != 0) | (avec != 0))
    nfull = oln // 16
    cp = nfull
    has_p = (oln - nfull * 16) != 0
    pf_u = jnp.where(use_u & has_p, jnp.full((16,), 1.0, jnp.float32), 0.0)
    tmask_p = (cp * 16 + iota < oln).astype(jnp.float32)
    po = pl.multiple_of(cp * 16, 16)
    plsc.store_compressed(ids_v.at[pl.ds(0, 16)], iota, mask=use_u)
    n_act = plsc.all_reduce_population_count(use_u)[0]

    def fire(k, _):
        kvec = jnp.full((16,), k, jnp.int32)
        uvec = plsc.load_gather(ids_v, [kvec])
        r = _take(rvec, uvec)[0]
        a = _take(avec, uvec)[0]
        off = pl.multiple_of(_take(xoff_u, uvec)[0], _T)
        vlab = jnp.where(uvec[0] < 8, vlab0, vlab1)
        pltpu.async_copy(srl_hbm.at[b, vlab, r],
                         rows_v.at[pl.ds(off, _T)], sem)
        pltpu.async_copy(vn_hbm.at[b, vlab, a],
                         rows_v.at[pl.ds(off + _L * _T, _T)], sem)
        return 0

    lax.fori_loop(0, n_act, fire, 0)


    def drain(i, _):
        pltpu.make_async_copy(srl_hbm.at[0, 0, 0],
                              rows_v.at[pl.ds(0, _T)], sem).wait()
        return 0

    lax.fori_loop(0, 2 * n_act, drain, 0)

    def u_body(k, tot):
        kvec = jnp.full((16,), k, jnp.int32)
        uvec = plsc.load_gather(ids_v, [kvec])
        trip = nfull
        rm = _take(rm_u, uvec)[0]
        am = _take(am_u, uvec)[0]
        pf = _take(pf_u, uvec)[0]
        xo = pl.multiple_of(_take(xoff_u, uvec)[0], _T)
        yo = xo + _L * _T

        def chunk2(c, acc):
            x0 = rows_v[pl.ds(xo + c * 32, 16)] * rm
            y0 = rows_v[pl.ds(yo + c * 32, 16)] * am
            x1 = rows_v[pl.ds(xo + c * 32 + 16, 16)] * rm
            y1 = rows_v[pl.ds(yo + c * 32 + 16, 16)] * am
            return acc + jnp.abs(x0 - y0) + jnp.abs(x1 - y1)

        tot = lax.fori_loop(0, trip // 2, chunk2, tot)
        lo = pl.multiple_of(jnp.maximum(trip - 1, 0) * 16, 16)
        lf = (trip - (trip // 2) * 2).astype(jnp.float32)
        xl = rows_v[pl.ds(xo + lo, 16)] * rm
        yl = rows_v[pl.ds(yo + lo, 16)] * am
        tot = tot + jnp.abs(xl - yl) * lf
        xp = rows_v[pl.ds(xo + po, 16)] * rm
        yp = rows_v[pl.ds(yo + po, 16)] * am
        return tot + jnp.abs(xp - yp) * (tmask_p * pf)

    total = lax.fori_loop(0, n_act, u_body, jnp.zeros((16,), jnp.float32))

    res_v[...] = total * inv_vec
    pltpu.sync_copy(res_v, out_hbm.at[wid])


def kernel(log_srl, log_vn, v_label, v_l, orig_l, semlink, semlink_l):
    srl_t = jnp.transpose(log_srl, (0, 1, 3, 2))
    vn_t = jnp.transpose(log_vn, (0, 1, 3, 2))
    sem_t = jnp.transpose(semlink.astype(jnp.int32), (0, 2, 3, 1))

    sc_call = pl.kernel(
        _sc_body,
        out_type=jax.ShapeDtypeStruct((_NW, 16), jnp.float32),
        mesh=plsc.VectorSubcoreMesh(core_axis_name="c", subcore_axis_name="s"),
        scratch_types=[
            pltpu.VMEM((_V,), jnp.int32),
            pltpu.VMEM((_V,), jnp.int32),
            pltpu.VMEM((2, _L, _V), jnp.int32),
            pltpu.VMEM((16,), jnp.int32),
            pltpu.VMEM((16,), jnp.int32),
            pltpu.VMEM((16,), jnp.int32),
            pltpu.VMEM((2 * 2 * _L * _T + 16,), jnp.float32),
            pltpu.VMEM((16,), jnp.float32),
            pltpu.SemaphoreType.DMA,
            pltpu.SemaphoreType.DMA,
        ],
        compiler_params=pltpu.CompilerParams(
            needs_layout_passes=False,
            use_tc_tiling_on_sc=True,
        ),
    )
    partials = sc_call(srl_t, vn_t,
                       v_label.astype(jnp.int32), semlink_l.astype(jnp.int32),
                       sem_t, v_l.astype(jnp.int32), orig_l.astype(jnp.int32))
    return jnp.sum(partials)


# speedup vs baseline: 1.0283x; 1.0020x over previous
import jax
import jax.numpy as jnp
from jax import lax
from jax.experimental import pallas as pl
from jax.experimental.pallas import tpu as pltpu
from jax.experimental.pallas import tpu_sc as plsc

_B, _T, _V, _L = 4, 256, 16, 8
_N = 40
_NC, _NS = 2, 16
_NW = _NC * _NS
_PAIRS_PER_W = (_B * _V) // _NW


_DNUMS = lax.GatherDimensionNumbers(
    offset_dims=(), collapsed_slice_dims=(0,), start_index_map=(0,))


def _take(vec, idx):
    return lax.gather(vec, idx[:, None], _DNUMS, (1,),
                      mode=lax.GatherScatterMode.PROMISE_IN_BOUNDS)


def _sc_body(srl_hbm, vn_hbm, vlab_hbm, sll_hbm, sem_hbm, vl_hbm, ol_hbm,
             out_hbm, vlab_v, sll_v, sem_v, vl_v, ol_v, ids_v, rows_v, res_v, sem, sem2):
    wid = lax.axis_index("s") * _NC + lax.axis_index("c")
    iota = lax.iota(jnp.int32, 16)
    pair0 = wid * _PAIRS_PER_W
    b = pair0 // _V
    v0 = pair0 - b * _V

    stage_a = [
        pltpu.async_copy(vlab_hbm.at[b], vlab_v, sem2),
        pltpu.async_copy(sem_hbm.at[b], sem_v, sem2),
    ]
    stage_b = [
        pltpu.async_copy(sll_hbm.at[b], sll_v, sem),
        pltpu.async_copy(vl_hbm, vl_v.at[pl.ds(0, _B)], sem),
        pltpu.async_copy(ol_hbm, ol_v.at[pl.ds(0, _B)], sem),
    ]
    for c in stage_a:
        c.wait()

    j_u = iota // 8
    l_u = iota - j_u * 8
    v0vec = jnp.full((16,), v0, jnp.int32)
    sr0 = plsc.load_gather(sem_v, [j_u, l_u, v0vec])
    sr1 = plsc.load_gather(sem_v, [j_u, l_u, v0vec + 1])
    vlab0 = plsc.load_gather(vlab_v, [v0vec])[0]
    vlab1 = plsc.load_gather(vlab_v, [v0vec + 1])[0]

    rvec = jnp.where(iota < 8, _take(sr0, l_u), _take(sr1, l_u))
    avec = jnp.where(iota < 8, _take(sr0, l_u + 8), _take(sr1, l_u + 8))
    rm_u = (rvec != 0).astype(jnp.float32)
    am_u = (avec != 0).astype(jnp.float32)
    xoff_u = (iota + j_u * 8) * _T

    for c in stage_b:
        c.wait()
    bvec = jnp.full((16,), b, jnp.int32)
    oln = plsc.load_gather(ol_v, [bvec])[0]
    vl_b = plsc.load_gather(vl_v, [bvec])[0]
    olv = ol_v[...]
    nrm = olv[0] + olv[1] + olv[2] + olv[3]
    inv_vec = 1.0 / jnp.full((16,), nrm.astype(jnp.float32))
    sll0 = plsc.load_gather(sll_v, [v0vec])[0]
    sll1 = plsc.load_gather(sll_v, [v0vec + 1])[0]
    sll_u = jnp.where(iota < 8, jnp.full((16,), sll0, jnp.int32),
                      jnp.full((16,), sll1, jnp.int32))
    use_u = (l_u < sll_u) & (v0 + j_u < vl_b) & ((rvec != 0) | (avec != 0))
    nfull = oln // 16
    cp = nfull
    has_p = (oln - nfull * 16) != 0
    pf_u = jnp.where(use_u & has_p, jnp.full((16,), 1.0, jnp.float32), 0.0)
    tmask_p = (cp * 16 + iota < oln).astype(jnp.float32)
    po = pl.multiple_of(cp * 16, 16)
    plsc.store_compressed(ids_v.at[pl.ds(0, 16)], iota, mask=use_u)
    n_act = plsc.all_reduce_population_count(use_u)[0]

    def fire(k, _):
        kvec = jnp.full((16,), k, jnp.int32)
        uvec = plsc.load_gather(ids_v, [kvec])
        r = _take(rvec, uvec)[0]
        a = _take(avec, uvec)[0]
        off = pl.multiple_of(_take(xoff_u, uvec)[0], _T)
        vlab = jnp.where(uvec[0] < 8, vlab0, vlab1)
        pltpu.async_copy(srl_hbm.at[b, vlab, r],
                         rows_v.at[pl.ds(off, _T)], sem)
        pltpu.async_copy(vn_hbm.at[b, vlab, a],
                         rows_v.at[pl.ds(off + _L * _T, _T)], sem)
        return 0

    lax.fori_loop(0, n_act, fire, 0)


    def drain(i, _):
        pltpu.make_async_copy(srl_hbm.at[0, 0, 0],
                              rows_v.at[pl.ds(0, _T)], sem).wait()
        return 0

    lax.fori_loop(0, 2 * n_act, drain, 0)

    def u_body(k, tot):
        kvec = jnp.full((16,), k, jnp.int32)
        uvec = plsc.load_gather(ids_v, [kvec])
        trip = nfull
        rm = _take(rm_u, uvec)[0]
        am = _take(am_u, uvec)[0]
        pf = _take(pf_u, uvec)[0]
        xo = pl.multiple_of(_take(xoff_u, uvec)[0], _T)
        yo = xo + _L * _T

        def chunk2(c, acc):
            x0 = rows_v[pl.ds(xo + c * 32, 16)] * rm
            y0 = rows_v[pl.ds(yo + c * 32, 16)] * am
            x1 = rows_v[pl.ds(xo + c * 32 + 16, 16)] * rm
            y1 = rows_v[pl.ds(yo + c * 32 + 16, 16)] * am
            return acc + jnp.abs(x0 - y0) + jnp.abs(x1 - y1)

        tot = lax.fori_loop(0, trip // 2, chunk2, tot)
        lo = pl.multiple_of(jnp.maximum(trip - 1, 0) * 16, 16)
        lf = (trip - (trip // 2) * 2).astype(jnp.float32)
        xl = rows_v[pl.ds(xo + lo, 16)] * rm
        yl = rows_v[pl.ds(yo + lo, 16)] * am
        tot = tot + jnp.abs(xl - yl) * lf
        xp = rows_v[pl.ds(xo + po, 16)] * rm
        yp = rows_v[pl.ds(yo + po, 16)] * am
        return tot + jnp.abs(xp - yp) * (tmask_p * pf)

    total = lax.fori_loop(0, n_act, u_body, jnp.zeros((16,), jnp.float32))

    res_v[...] = total * inv_vec
    pltpu.sync_copy(res_v, out_hbm.at[wid])


def kernel(log_srl, log_vn, v_label, v_l, orig_l, semlink, semlink_l):
    srl_t = jnp.transpose(log_srl, (0, 1, 3, 2))
    vn_t = jnp.transpose(log_vn, (0, 1, 3, 2))
    sem_t = jnp.transpose(semlink.astype(jnp.int32), (0, 2, 3, 1))

    sc_call = pl.kernel(
        _sc_body,
        out_type=jax.ShapeDtypeStruct((_NW, 16), jnp.float32),
        mesh=plsc.VectorSubcoreMesh(core_axis_name="c", subcore_axis_name="s"),
        scratch_types=[
            pltpu.VMEM((_V,), jnp.int32),
            pltpu.VMEM((_V,), jnp.int32),
            pltpu.VMEM((2, _L, _V), jnp.int32),
            pltpu.VMEM((16,), jnp.int32),
            pltpu.VMEM((16,), jnp.int32),
            pltpu.VMEM((16,), jnp.int32),
            pltpu.VMEM((2 * 2 * _L * _T + 16,), jnp.float32),
            pltpu.VMEM((16,), jnp.float32),
            pltpu.SemaphoreType.DMA,
            pltpu.SemaphoreType.DMA,
        ],
        compiler_params=pltpu.CompilerParams(
            needs_layout_passes=False,
            use_tc_tiling_on_sc=True,
            skip_device_barrier=True,
        ),
    )
    partials = sc_call(srl_t, vn_t,
                       v_label.astype(jnp.int32), semlink_l.astype(jnp.int32),
                       sem_t, v_l.astype(jnp.int32), orig_l.astype(jnp.int32))
    return jnp.sum(partials)
